# baseline (speedup 1.0000x reference)
import jax
import jax.numpy as jnp
from jax.experimental import pallas as pl
from jax.experimental.pallas import tpu as pltpu

_B, _P, _D = 64, 1024, 768


def _body(pos_ref, vis_ref, tab_ref, out_ref, emb_ref):
    b = pl.program_id(0)

    @pl.when(b == 0)
    def _gather():
        def row(i, carry):
            idx = pos_ref[i]
            emb_ref[pl.ds(i, 1), :] = tab_ref[pl.ds(idx, 1), :]
            return carry

        jax.lax.fori_loop(0, _P, row, 0)

    out_ref[...] = vis_ref[...] + emb_ref[...][None, :, :]


def kernel(visual_tokens, pos_table, positions):
    grid_spec = pltpu.PrefetchScalarGridSpec(
        num_scalar_prefetch=1,
        grid=(_B,),
        in_specs=[
            pl.BlockSpec((1, _P, _D), lambda b, pos: (b, 0, 0)),
            pl.BlockSpec((_P, _D), lambda b, pos: (0, 0)),
        ],
        out_specs=pl.BlockSpec((1, _P, _D), lambda b, pos: (b, 0, 0)),
        scratch_shapes=[pltpu.VMEM((_P, _D), jnp.float32)],
    )
    return pl.pallas_call(
        _body,
        grid_spec=grid_spec,
        out_shape=jax.ShapeDtypeStruct((_B, _P, _D), jnp.float32),
    )(positions, visual_tokens, pos_table)


# speedup vs baseline: 1.0972x; 1.0972x over previous
import jax
import jax.numpy as jnp
from jax.experimental import pallas as pl
from jax.experimental.pallas import tpu as pltpu

_B, _P, _D = 64, 1024, 768
_BB = 2


def _body(pos_sref, vis_ref, tab_ref, posv_ref, out_ref, emb_ref):
    b = pl.program_id(0)
    iota = jax.lax.broadcasted_iota(jnp.int32, (1, _P), 1)
    ident = jnp.all(posv_ref[...] == iota)

    @pl.when(jnp.logical_and(b == 0, jnp.logical_not(ident)))
    def _gather():
        def row(i, carry):
            emb_ref[pl.ds(i, 1), :] = tab_ref[pl.ds(pos_sref[i], 1), :]
            return carry

        jax.lax.fori_loop(0, _P, row, 0)

    @pl.when(ident)
    def _fast():
        out_ref[...] = vis_ref[...] + tab_ref[...][None, :, :]

    @pl.when(jnp.logical_not(ident))
    def _slow():
        out_ref[...] = vis_ref[...] + emb_ref[...][None, :, :]


def kernel(visual_tokens, pos_table, positions):
    grid_spec = pltpu.PrefetchScalarGridSpec(
        num_scalar_prefetch=1,
        grid=(_B // _BB,),
        in_specs=[
            pl.BlockSpec((_BB, _P, _D), lambda b, pos: (b, 0, 0)),
            pl.BlockSpec((_P, _D), lambda b, pos: (0, 0)),
            pl.BlockSpec((1, _P), lambda b, pos: (0, 0)),
        ],
        out_specs=pl.BlockSpec((_BB, _P, _D), lambda b, pos: (b, 0, 0)),
        scratch_shapes=[pltpu.VMEM((_P, _D), jnp.float32)],
    )
    return pl.pallas_call(
        _body,
        grid_spec=grid_spec,
        out_shape=jax.ShapeDtypeStruct((_B, _P, _D), jnp.float32),
    )(positions, visual_tokens, pos_table, positions.reshape(1, _P))


# speedup vs baseline: 1.1088x; 1.0106x over previous
import jax
import jax.numpy as jnp
from jax.experimental import pallas as pl
from jax.experimental.pallas import tpu as pltpu

_B, _P, _D = 64, 1024, 768
_BB = 4


def _body(pos_sref, vis_ref, tab_ref, posv_ref, out_ref, emb_ref):
    b = pl.program_id(0)
    iota = jax.lax.broadcasted_iota(jnp.int32, (1, _P), 1)
    ident = jnp.all(posv_ref[...] == iota)

    @pl.when(jnp.logical_and(b == 0, jnp.logical_not(ident)))
    def _gather():
        def row(i, carry):
            emb_ref[pl.ds(i, 1), :] = tab_ref[pl.ds(pos_sref[i], 1), :]
            return carry

        jax.lax.fori_loop(0, _P, row, 0)

    @pl.when(ident)
    def _fast():
        out_ref[...] = vis_ref[...] + tab_ref[...][None, :, :]

    @pl.when(jnp.logical_not(ident))
    def _slow():
        out_ref[...] = vis_ref[...] + emb_ref[...][None, :, :]


def kernel(visual_tokens, pos_table, positions):
    grid_spec = pltpu.PrefetchScalarGridSpec(
        num_scalar_prefetch=1,
        grid=(_B // _BB,),
        in_specs=[
            pl.BlockSpec((_BB, _P, _D), lambda b, pos: (b, 0, 0)),
            pl.BlockSpec((_P, _D), lambda b, pos: (0, 0)),
            pl.BlockSpec((1, _P), lambda b, pos: (0, 0)),
        ],
        out_specs=pl.BlockSpec((_BB, _P, _D), lambda b, pos: (b, 0, 0)),
        scratch_shapes=[pltpu.VMEM((_P, _D), jnp.float32)],
    )
    return pl.pallas_call(
        _body,
        grid_spec=grid_spec,
        out_shape=jax.ShapeDtypeStruct((_B, _P, _D), jnp.float32),
    )(positions, visual_tokens, pos_table, positions.reshape(1, _P))
